# Initial kernel scaffold; baseline (speedup 1.0000x reference)
import jax
import jax.numpy as jnp
from jax.experimental import pallas as pl


def kernel(x, W_base, b_base, A, B, Wr):
    raise NotImplementedError("write your pallas kernel here")


# speedup vs baseline: 4.6577x; 4.6577x over previous
import functools

import jax
import jax.numpy as jnp
from jax.experimental import pallas as pl

T = 4096
D = 1024
OUT = 1024
R = 64
E = 16
K = 2
TB = 256


def _fused_body(x_ref, wbt_ref, b_ref, at_ref, b2_ref, wrt_ref, o_ref):
    x = x_ref[...]
    h = jnp.dot(x, at_ref[...], preferred_element_type=jnp.float32)
    logits = jnp.dot(x, wrt_ref[...], preferred_element_type=jnp.float32)
    w = jax.nn.softmax(logits, axis=-1)
    eids = jax.lax.broadcasted_iota(jnp.int32, w.shape, 1)
    i1 = jnp.argmax(w, axis=-1)
    w1 = jnp.max(w, axis=-1)
    masked = jnp.where(eids == i1[:, None], -jnp.inf, w)
    i2 = jnp.argmax(masked, axis=-1)
    w2 = jnp.max(masked, axis=-1)
    s = w1 + w2 + 1e-6
    w_eff = (jnp.where(eids == i1[:, None], w1[:, None], 0.0)
             + jnp.where(eids == i2[:, None], w2[:, None], 0.0)) / s[:, None]
    p = (w_eff[:, :, None] * h[:, None, :]).reshape(TB, E * R)
    acc = jnp.dot(x, wbt_ref[...], preferred_element_type=jnp.float32)
    acc = acc + jnp.dot(p, b2_ref[...], preferred_element_type=jnp.float32)
    o_ref[...] = acc + b_ref[...]


@functools.partial(jax.jit, static_argnames=())
def kernel(x, W_base, b_base, A, B, Wr):
    wbt = W_base.T
    at = A.T
    wrt = Wr.T
    b2 = B.transpose(0, 2, 1).reshape(E * R, OUT)
    b2d = b_base.reshape(1, OUT)
    grid = (T // TB,)
    return pl.pallas_call(
        _fused_body,
        grid=grid,
        in_specs=[
            pl.BlockSpec((TB, D), lambda i: (i, 0)),
            pl.BlockSpec((D, OUT), lambda i: (0, 0)),
            pl.BlockSpec((1, OUT), lambda i: (0, 0)),
            pl.BlockSpec((D, R), lambda i: (0, 0)),
            pl.BlockSpec((E * R, OUT), lambda i: (0, 0)),
            pl.BlockSpec((D, E), lambda i: (0, 0)),
        ],
        out_specs=pl.BlockSpec((TB, OUT), lambda i: (i, 0)),
        out_shape=jax.ShapeDtypeStruct((T, OUT), jnp.float32),
    )(x, wbt, b2d, at, b2, wrt)


# speedup vs baseline: 4.6796x; 1.0047x over previous
import functools

import jax
import jax.numpy as jnp
from jax.experimental import pallas as pl

T = 4096
D = 1024
OUT = 1024
R = 64
E = 16
K = 2
TB = 256


def _fused_body(x_ref, wbt_ref, b_ref, at_ref, b2_ref, wrt_ref, o_ref):
    x = x_ref[...]
    xb = x.astype(jnp.bfloat16)
    h = jnp.dot(xb, at_ref[...], preferred_element_type=jnp.float32)
    logits = jnp.dot(x, wrt_ref[...], preferred_element_type=jnp.float32)
    w = jax.nn.softmax(logits, axis=-1)
    eids = jax.lax.broadcasted_iota(jnp.int32, w.shape, 1)
    i1 = jnp.argmax(w, axis=-1)
    w1 = jnp.max(w, axis=-1)
    masked = jnp.where(eids == i1[:, None], -jnp.inf, w)
    i2 = jnp.argmax(masked, axis=-1)
    w2 = jnp.max(masked, axis=-1)
    s = w1 + w2 + 1e-6
    w_eff = (jnp.where(eids == i1[:, None], w1[:, None], 0.0)
             + jnp.where(eids == i2[:, None], w2[:, None], 0.0)) / s[:, None]
    p = (w_eff[:, :, None] * h[:, None, :]).reshape(TB, E * R).astype(jnp.bfloat16)
    acc = jnp.dot(xb, wbt_ref[...], preferred_element_type=jnp.float32)
    acc = acc + jnp.dot(p, b2_ref[...], preferred_element_type=jnp.float32)
    o_ref[...] = acc + b_ref[...]


@functools.partial(jax.jit, static_argnames=())
def kernel(x, W_base, b_base, A, B, Wr):
    wbt = W_base.T.astype(jnp.bfloat16)
    at = A.T.astype(jnp.bfloat16)
    wrt = Wr.T
    b2 = B.transpose(0, 2, 1).reshape(E * R, OUT).astype(jnp.bfloat16)
    b2d = b_base.reshape(1, OUT)
    grid = (T // TB,)
    return pl.pallas_call(
        _fused_body,
        grid=grid,
        in_specs=[
            pl.BlockSpec((TB, D), lambda i: (i, 0)),
            pl.BlockSpec((D, OUT), lambda i: (0, 0)),
            pl.BlockSpec((1, OUT), lambda i: (0, 0)),
            pl.BlockSpec((D, R), lambda i: (0, 0)),
            pl.BlockSpec((E * R, OUT), lambda i: (0, 0)),
            pl.BlockSpec((D, E), lambda i: (0, 0)),
        ],
        out_specs=pl.BlockSpec((TB, OUT), lambda i: (i, 0)),
        out_shape=jax.ShapeDtypeStruct((T, OUT), jnp.float32),
    )(x, wbt, b2d, at, b2, wrt)
